# Initial kernel scaffold; baseline (speedup 1.0000x reference)
import jax
import jax.numpy as jnp
from jax.experimental import pallas as pl


def kernel(x, W):
    raise NotImplementedError("write your pallas kernel here")


# speedup vs baseline: 4.9213x; 4.9213x over previous
import functools

import jax
import jax.numpy as jnp
from jax import lax
from jax.experimental import pallas as pl
from jax.experimental.pallas import tpu as pltpu
from jax.experimental.pallas import tpu_sc as plsc

BATCH = 16384
HIST = 200
EMBED_DIM = 3
PAD_DIM = 8
N_TOTAL = BATCH * HIST
NC, NS = 2, 16
NW = NC * NS
PER_W = N_TOTAL // NW
CHUNK = 10240

_mesh = plsc.VectorSubcoreMesh(core_axis_name="c", subcore_axis_name="s")


@functools.partial(
    pl.kernel,
    mesh=_mesh,
    out_type=jax.ShapeDtypeStruct((N_TOTAL, EMBED_DIM), jnp.float32),
    scratch_types=[
        pltpu.VMEM((CHUNK,), jnp.int32),
        pltpu.VMEM((CHUNK, PAD_DIM), jnp.float32),
        pltpu.SemaphoreType.DMA,
    ],
    compiler_params=pltpu.CompilerParams(use_tc_tiling_on_sc=False),
)
def _gather(idx_hbm, w_hbm, out_hbm, idx_v, rows_v, sem):
    wid = lax.axis_index("s") * NC + lax.axis_index("c")
    base = wid * PER_W

    def body(j, carry):
        start = base + j * CHUNK
        pltpu.sync_copy(idx_hbm.at[pl.ds(start, CHUNK)], idx_v)
        pltpu.async_copy(w_hbm.at[idx_v], rows_v, sem).wait()
        pltpu.sync_copy(rows_v.at[:, 0:EMBED_DIM], out_hbm.at[pl.ds(start, CHUNK)])
        return carry

    lax.fori_loop(0, PER_W // CHUNK, body, 0)


def kernel(x, W):
    w_pad = jnp.pad(W, ((0, 0), (0, PAD_DIM - EMBED_DIM)))
    out = _gather(x.reshape(N_TOTAL), w_pad)
    return out.reshape(BATCH, HIST, EMBED_DIM)


# speedup vs baseline: 111.5127x; 22.6593x over previous
import functools

import jax
import jax.numpy as jnp
from jax import lax
from jax.experimental import pallas as pl
from jax.experimental.pallas import tpu as pltpu
from jax.experimental.pallas import tpu_sc as plsc

BATCH = 16384
HIST = 200
EMBED_DIM = 3
N_VOCAB = 1000000
N_TOTAL = BATCH * HIST
NC, NS = 2, 16
NW = NC * NS
PER_W = N_TOTAL // NW
CHUNK = 10240

_mesh = plsc.VectorSubcoreMesh(core_axis_name="c", subcore_axis_name="s")


@functools.partial(
    pl.kernel,
    mesh=_mesh,
    out_type=jax.ShapeDtypeStruct((EMBED_DIM * N_TOTAL,), jnp.float32),
    scratch_types=[
        pltpu.VMEM((CHUNK,), jnp.int32),
        pltpu.VMEM((EMBED_DIM, CHUNK), jnp.float32),
        pltpu.SemaphoreType.DMA,
    ],
    compiler_params=pltpu.CompilerParams(use_tc_tiling_on_sc=False),
)
def _gather(idx_hbm, wt_hbm, out_hbm, idx_v, buf_v, sem):
    wid = lax.axis_index("s") * NC + lax.axis_index("c")
    base = wid * PER_W

    def body(j, carry):
        start = base + j * CHUNK
        pltpu.sync_copy(idx_hbm.at[pl.ds(start, CHUNK)], idx_v)
        cps = [
            pltpu.async_copy(wt_hbm.at[d].at[idx_v], buf_v.at[d], sem)
            for d in range(EMBED_DIM)
        ]
        for cp in cps:
            cp.wait()
        for d in range(EMBED_DIM):
            pltpu.sync_copy(
                buf_v.at[d], out_hbm.at[pl.ds(d * N_TOTAL + start, CHUNK)]
            )
        return carry

    lax.fori_loop(0, PER_W // CHUNK, body, 0)


def _asm_body(in_ref, out_ref):
    out_ref[0] = in_ref[...].reshape(8, BATCH)


_assemble = pl.pallas_call(
    _asm_body,
    grid=(EMBED_DIM, HIST // 8),
    in_specs=[pl.BlockSpec((8 * BATCH,), lambda d, r: (d * (HIST // 8) + r,))],
    out_specs=pl.BlockSpec((1, 8, BATCH), lambda d, r: (d, r, 0)),
    out_shape=jax.ShapeDtypeStruct((EMBED_DIM, HIST, BATCH), jnp.float32),
)


def kernel(x, W):
    x1f = jnp.transpose(x).reshape(N_TOTAL)
    wt = jnp.transpose(W).reshape(EMBED_DIM, N_VOCAB)
    planes = _gather(x1f, wt)
    asm = _assemble(planes)
    return jnp.transpose(asm, (2, 1, 0))


# speedup vs baseline: 174.8607x; 1.5681x over previous
import functools

import jax
import jax.numpy as jnp
from jax import lax
from jax.experimental import pallas as pl
from jax.experimental.pallas import tpu as pltpu
from jax.experimental.pallas import tpu_sc as plsc

BATCH = 16384
HIST = 200
EMBED_DIM = 3
PAD_DIM = 8
N_VOCAB = 1000000
N_TOTAL = BATCH * HIST
NC, NS = 2, 16
NW = NC * NS
PER_W = N_TOTAL // NW
CHUNK = 5120
NCH = PER_W // CHUNK

VC = 6400
NVCH = N_VOCAB // VC
VTAIL = N_VOCAB - NVCH * VC

_mesh = plsc.VectorSubcoreMesh(core_axis_name="c", subcore_axis_name="s")
_params = pltpu.CompilerParams(
    use_tc_tiling_on_sc=False, needs_layout_passes=False
)


@functools.partial(
    pl.kernel,
    mesh=_mesh,
    out_type=jax.ShapeDtypeStruct((PAD_DIM * N_VOCAB,), jnp.float32),
    scratch_types=[
        pltpu.VMEM((VC,), jnp.float32),
        pltpu.VMEM((VC,), jnp.float32),
        pltpu.VMEM((VC,), jnp.float32),
        pltpu.VMEM((PAD_DIM * VC,), jnp.float32),
        pltpu.SemaphoreType.DMA,
    ],
    compiler_params=_params,
)
def _prep(wt_hbm, w8_hbm, p0, p1, p2, w8v, sem):
    wid = lax.axis_index("s") * NC + lax.axis_index("c")
    lanes = lax.iota(jnp.int32, 16)
    q8 = 8 * lanes
    planes = (p0, p1, p2)

    def do_chunk(v0, size):
        for d in range(EMBED_DIM):
            pltpu.sync_copy(
                wt_hbm.at[pl.ds(d * N_VOCAB + v0, size)],
                planes[d].at[pl.ds(0, size)],
            )

        def body(t, carry):
            for d in range(EMBED_DIM):
                vals = planes[d][pl.ds(16 * t, 16)]
                plsc.store_scatter(w8v, [q8 + (128 * t + d)], vals)
            return carry

        lax.fori_loop(0, size // 16, body, 0)
        pltpu.sync_copy(
            w8v.at[pl.ds(0, size * PAD_DIM)],
            w8_hbm.at[pl.ds(v0 * PAD_DIM, size * PAD_DIM)],
        )

    for i in range(5):
        k = wid + NW * i

        @pl.when(k < NVCH)
        def _():
            do_chunk(pl.multiple_of(k * VC, 8), VC)

    @pl.when(wid == 0)
    def _():
        do_chunk(NVCH * VC, VTAIL)


@functools.partial(
    pl.kernel,
    mesh=_mesh,
    out_type=jax.ShapeDtypeStruct((EMBED_DIM * N_TOTAL,), jnp.float32),
    scratch_types=[
        pltpu.VMEM((CHUNK,), jnp.int32),
        pltpu.VMEM((CHUNK,), jnp.int32),
        pltpu.VMEM((CHUNK, PAD_DIM), jnp.float32),
        pltpu.VMEM((CHUNK, PAD_DIM), jnp.float32),
        pltpu.VMEM((EMBED_DIM, CHUNK), jnp.float32),
        pltpu.SemaphoreType.DMA,
        pltpu.SemaphoreType.DMA,
        pltpu.SemaphoreType.DMA,
    ],
    compiler_params=_params,
)
def _gather(idx_hbm, w8_hbm, out_hbm, idxA, idxB, rowsA, rowsB, pbuf,
            semI, semGA, semGB):
    wid = lax.axis_index("s") * NC + lax.axis_index("c")
    base = wid * PER_W
    lanes = lax.iota(jnp.int32, 16)
    idx_bufs = (idxA, idxB)
    row_bufs = (rowsA, rowsB)
    row_sems = (semGA, semGB)
    dcols = [jnp.full((16,), d, jnp.int32) for d in range(EMBED_DIM)]

    def extract_and_store(rows, j):
        start = base + j * CHUNK

        def body(t, carry):
            rid = 16 * t + lanes
            for d in range(EMBED_DIM):
                pbuf[d, pl.ds(16 * t, 16)] = plsc.load_gather(
                    rows, [rid, dcols[d]]
                )
            return carry

        lax.fori_loop(0, CHUNK // 16, body, 0)
        for d in range(EMBED_DIM):
            pltpu.sync_copy(
                pbuf.at[d], out_hbm.at[pl.ds(d * N_TOTAL + start, CHUNK)]
            )

    pltpu.async_copy(idx_hbm.at[pl.ds(base, CHUNK)], idxA, semI)
    for j in range(NCH):
        b = j & 1
        pltpu.make_async_copy(
            idx_hbm.at[pl.ds(base, CHUNK)], idx_bufs[b], semI
        ).wait()
        if j > 0:
            pltpu.make_async_copy(
                w8_hbm.at[pl.ds(0, CHUNK)], row_bufs[1 - b], row_sems[1 - b]
            ).wait()
        pltpu.async_copy(w8_hbm.at[idx_bufs[b]], row_bufs[b], row_sems[b])
        if j + 1 < NCH:
            pltpu.async_copy(
                idx_hbm.at[pl.ds(base + (j + 1) * CHUNK, CHUNK)],
                idx_bufs[1 - b], semI,
            )
        if j > 0:
            extract_and_store(row_bufs[1 - b], j - 1)
    bl = (NCH - 1) & 1
    pltpu.make_async_copy(
        w8_hbm.at[pl.ds(0, CHUNK)], row_bufs[bl], row_sems[bl]
    ).wait()
    extract_and_store(row_bufs[bl], NCH - 1)


def _asm_body(in_ref, out_ref):
    out_ref[0] = in_ref[...].reshape(8, BATCH)


_assemble = pl.pallas_call(
    _asm_body,
    grid=(EMBED_DIM, HIST // 8),
    in_specs=[pl.BlockSpec((8 * BATCH,), lambda d, r: (d * (HIST // 8) + r,))],
    out_specs=pl.BlockSpec((1, 8, BATCH), lambda d, r: (d, r, 0)),
    out_shape=jax.ShapeDtypeStruct((EMBED_DIM, HIST, BATCH), jnp.float32),
)


def kernel(x, W):
    x1f = jnp.transpose(x).reshape(N_TOTAL)
    wt = jnp.transpose(W).reshape(EMBED_DIM * N_VOCAB)
    w8 = _prep(wt).reshape(N_VOCAB, PAD_DIM)
    planes = _gather(x1f, w8)
    asm = _assemble(planes)
    return jnp.transpose(asm, (2, 1, 0))


# speedup vs baseline: 177.1708x; 1.0132x over previous
import functools

import jax
import jax.numpy as jnp
from jax import lax
from jax.experimental import pallas as pl
from jax.experimental.pallas import tpu as pltpu
from jax.experimental.pallas import tpu_sc as plsc

BATCH = 16384
HIST = 200
EMBED_DIM = 3
PAD_DIM = 8
N_VOCAB = 1000000
N_TOTAL = BATCH * HIST
NC, NS = 2, 16
NW = NC * NS
PER_W = N_TOTAL // NW
CHUNK = 5120
NCH = PER_W // CHUNK

VC = 6400
NVCH = N_VOCAB // VC
VTAIL = N_VOCAB - NVCH * VC

_mesh = plsc.VectorSubcoreMesh(core_axis_name="c", subcore_axis_name="s")
_params = pltpu.CompilerParams(
    use_tc_tiling_on_sc=False, needs_layout_passes=False
)


@functools.partial(
    pl.kernel,
    mesh=_mesh,
    out_type=jax.ShapeDtypeStruct((PAD_DIM * N_VOCAB,), jnp.float32),
    scratch_types=[
        pltpu.VMEM((VC,), jnp.float32),
        pltpu.VMEM((VC,), jnp.float32),
        pltpu.VMEM((VC,), jnp.float32),
        pltpu.VMEM((PAD_DIM * VC,), jnp.float32),
        pltpu.SemaphoreType.DMA,
    ],
    compiler_params=_params,
)
def _prep(wt_hbm, w8_hbm, p0, p1, p2, w8v, sem):
    wid = lax.axis_index("s") * NC + lax.axis_index("c")
    lanes = lax.iota(jnp.int32, 16)
    q8 = 8 * lanes
    planes = (p0, p1, p2)

    def do_chunk(v0, size):
        for d in range(EMBED_DIM):
            pltpu.sync_copy(
                wt_hbm.at[pl.ds(d * N_VOCAB + v0, size)],
                planes[d].at[pl.ds(0, size)],
            )

        def body(t, carry):
            for d in range(EMBED_DIM):
                vals = planes[d][pl.ds(16 * t, 16)]
                plsc.store_scatter(w8v, [q8 + (128 * t + d)], vals)
            return carry

        lax.fori_loop(0, size // 16, body, 0)
        pltpu.sync_copy(
            w8v.at[pl.ds(0, size * PAD_DIM)],
            w8_hbm.at[pl.ds(v0 * PAD_DIM, size * PAD_DIM)],
        )

    for i in range(5):
        k = wid + NW * i

        @pl.when(k < NVCH)
        def _():
            do_chunk(pl.multiple_of(k * VC, 8), VC)

    @pl.when(wid == 0)
    def _():
        do_chunk(NVCH * VC, VTAIL)


@functools.partial(
    pl.kernel,
    mesh=_mesh,
    out_type=jax.ShapeDtypeStruct((EMBED_DIM * N_TOTAL,), jnp.float32),
    scratch_types=[
        pltpu.VMEM((CHUNK,), jnp.int32),
        pltpu.VMEM((CHUNK,), jnp.int32),
        pltpu.VMEM((CHUNK, PAD_DIM), jnp.float32),
        pltpu.VMEM((CHUNK, PAD_DIM), jnp.float32),
        pltpu.VMEM((EMBED_DIM, CHUNK), jnp.float32),
        pltpu.SemaphoreType.DMA,
        pltpu.SemaphoreType.DMA,
        pltpu.SemaphoreType.DMA,
    ],
    compiler_params=_params,
)
def _gather(idx_hbm, w8_hbm, out_hbm, idxA, idxB, rowsA, rowsB, pbuf,
            semI, semGA, semGB):
    wid = lax.axis_index("s") * NC + lax.axis_index("c")
    base = wid * PER_W
    lanes = lax.iota(jnp.int32, 16)
    idx_bufs = (idxA, idxB)
    row_bufs = (rowsA, rowsB)
    row_sems = (semGA, semGB)
    dcols = [jnp.full((16,), d, jnp.int32) for d in range(EMBED_DIM)]

    def extract_and_store(rows, j):
        start = base + j * CHUNK

        def body(t, carry):
            rid = 16 * t + lanes
            for d in range(EMBED_DIM):
                pbuf[d, pl.ds(16 * t, 16)] = plsc.load_gather(
                    rows, [rid, dcols[d]]
                )
            return carry

        lax.fori_loop(0, CHUNK // 16, body, 0)
        for d in range(EMBED_DIM):
            pltpu.sync_copy(
                pbuf.at[d], out_hbm.at[pl.ds(d * N_TOTAL + start, CHUNK)]
            )

    pltpu.async_copy(idx_hbm.at[pl.ds(base, CHUNK)], idxA, semI)
    for j in range(NCH):
        b = j & 1
        pltpu.make_async_copy(
            idx_hbm.at[pl.ds(base, CHUNK)], idx_bufs[b], semI
        ).wait()
        if j > 0:
            pltpu.make_async_copy(
                w8_hbm.at[pl.ds(0, CHUNK)], row_bufs[1 - b], row_sems[1 - b]
            ).wait()
        pltpu.async_copy(w8_hbm.at[idx_bufs[b]], row_bufs[b], row_sems[b])
        if j + 1 < NCH:
            pltpu.async_copy(
                idx_hbm.at[pl.ds(base + (j + 1) * CHUNK, CHUNK)],
                idx_bufs[1 - b], semI,
            )
        if j > 0:
            extract_and_store(row_bufs[1 - b], j - 1)
    bl = (NCH - 1) & 1
    pltpu.make_async_copy(
        w8_hbm.at[pl.ds(0, CHUNK)], row_bufs[bl], row_sems[bl]
    ).wait()
    extract_and_store(row_bufs[bl], NCH - 1)


def _asm_body(in_ref, out_ref):
    a = in_ref[...].reshape(128, 8, 128)
    a = jnp.transpose(a, (1, 0, 2))
    out_ref[0] = a.reshape(8, BATCH)


_assemble = pl.pallas_call(
    _asm_body,
    grid=(EMBED_DIM, HIST // 8),
    in_specs=[pl.BlockSpec((8 * BATCH,), lambda d, r: (d * (HIST // 8) + r,))],
    out_specs=pl.BlockSpec((1, 8, BATCH), lambda d, r: (d, r, 0)),
    out_shape=jax.ShapeDtypeStruct((EMBED_DIM, HIST, BATCH), jnp.float32),
)


def kernel(x, W):
    x1f = (jnp.transpose(x).reshape(HIST // 8, 8, BATCH // 128, 128)
           .transpose(0, 2, 1, 3).reshape(N_TOTAL))
    wt = jnp.transpose(W).reshape(EMBED_DIM * N_VOCAB)
    w8 = _prep(wt).reshape(N_VOCAB, PAD_DIM)
    planes = _gather(x1f, w8)
    asm = _assemble(planes)
    return jnp.transpose(asm, (2, 1, 0))


# speedup vs baseline: 183.7174x; 1.0370x over previous
import functools

import jax
import jax.numpy as jnp
from jax import lax
from jax.experimental import pallas as pl
from jax.experimental.pallas import tpu as pltpu
from jax.experimental.pallas import tpu_sc as plsc

BATCH = 16384
HIST = 200
EMBED_DIM = 3
PAD_DIM = 8
N_VOCAB = 1000000
N_TOTAL = BATCH * HIST
NC, NS = 2, 16
NW = NC * NS
PER_W = N_TOTAL // NW
CHUNK = 5120
NCH = PER_W // CHUNK

VC = 6400
NVCH = N_VOCAB // VC
VTAIL = N_VOCAB - NVCH * VC

_mesh = plsc.VectorSubcoreMesh(core_axis_name="c", subcore_axis_name="s")
_params = pltpu.CompilerParams(
    use_tc_tiling_on_sc=False, needs_layout_passes=False
)


@functools.partial(
    pl.kernel,
    mesh=_mesh,
    out_type=jax.ShapeDtypeStruct((PAD_DIM * N_VOCAB,), jnp.float32),
    scratch_types=[
        pltpu.VMEM((VC,), jnp.float32),
        pltpu.VMEM((VC,), jnp.float32),
        pltpu.VMEM((VC,), jnp.float32),
        pltpu.VMEM((PAD_DIM * VC,), jnp.float32),
        pltpu.SemaphoreType.DMA,
    ],
    compiler_params=_params,
)
def _prep(wt_hbm, w8_hbm, p0, p1, p2, w8v, sem):
    wid = lax.axis_index("s") * NC + lax.axis_index("c")
    lanes = lax.iota(jnp.int32, 16)
    q8 = 8 * lanes
    planes = (p0, p1, p2)

    def issue_in(k):
        @pl.when(k < NVCH)
        def _():
            v0 = pl.multiple_of(k * VC, 8)
            for d in range(EMBED_DIM):
                pltpu.async_copy(
                    wt_hbm.at[pl.ds(d * N_VOCAB + v0, VC)], planes[d], sem
                )

    def scatter(size):
        def body(t, carry):
            for d in range(EMBED_DIM):
                vals = planes[d][pl.ds(16 * t, 16)]
                plsc.store_scatter(w8v, [q8 + (128 * t + d)], vals)
            return carry

        lax.fori_loop(0, size // 16, body, 0)

    issue_in(wid)
    for i in range(5):
        k = wid + NW * i

        @pl.when(k < NVCH)
        def _():
            for d in range(EMBED_DIM):
                pltpu.make_async_copy(
                    wt_hbm.at[pl.ds(0, VC)], planes[d], sem
                ).wait()
            scatter(VC)
            issue_in(k + NW)
            v0 = pl.multiple_of(k * VC, 8)
            pltpu.sync_copy(w8v, w8_hbm.at[pl.ds(v0 * PAD_DIM, VC * PAD_DIM)])

    @pl.when(wid == 0)
    def _():
        v0 = NVCH * VC
        for d in range(EMBED_DIM):
            pltpu.sync_copy(
                wt_hbm.at[pl.ds(d * N_VOCAB + v0, VTAIL)],
                planes[d].at[pl.ds(0, VTAIL)],
            )
        scatter(VTAIL)
        pltpu.sync_copy(
            w8v.at[pl.ds(0, VTAIL * PAD_DIM)],
            w8_hbm.at[pl.ds(v0 * PAD_DIM, VTAIL * PAD_DIM)],
        )


@functools.partial(
    pl.kernel,
    mesh=_mesh,
    out_type=jax.ShapeDtypeStruct((EMBED_DIM * N_TOTAL,), jnp.float32),
    scratch_types=[
        pltpu.VMEM((CHUNK,), jnp.int32),
        pltpu.VMEM((CHUNK,), jnp.int32),
        pltpu.VMEM((CHUNK, PAD_DIM), jnp.float32),
        pltpu.VMEM((CHUNK, PAD_DIM), jnp.float32),
        pltpu.VMEM((EMBED_DIM, CHUNK), jnp.float32),
        pltpu.SemaphoreType.DMA,
        pltpu.SemaphoreType.DMA,
        pltpu.SemaphoreType.DMA,
    ],
    compiler_params=_params,
)
def _gather(idx_hbm, w8_hbm, out_hbm, idxA, idxB, rowsA, rowsB, pbuf,
            semI, semGA, semGB):
    wid = lax.axis_index("s") * NC + lax.axis_index("c")
    base = wid * PER_W
    lanes = lax.iota(jnp.int32, 16)
    idx_bufs = (idxA, idxB)
    row_bufs = (rowsA, rowsB)
    row_sems = (semGA, semGB)
    dcols = [jnp.full((16,), d, jnp.int32) for d in range(EMBED_DIM)]

    def extract_and_store(rows, j):
        start = base + j * CHUNK

        def body(t, carry):
            rid = 16 * t + lanes
            for d in range(EMBED_DIM):
                pbuf[d, pl.ds(16 * t, 16)] = plsc.load_gather(
                    rows, [rid, dcols[d]]
                )
            return carry

        lax.fori_loop(0, CHUNK // 16, body, 0)
        for d in range(EMBED_DIM):
            pltpu.sync_copy(
                pbuf.at[d], out_hbm.at[pl.ds(d * N_TOTAL + start, CHUNK)]
            )

    pltpu.async_copy(idx_hbm.at[pl.ds(base, CHUNK)], idxA, semI)
    for j in range(NCH):
        b = j & 1
        pltpu.make_async_copy(
            idx_hbm.at[pl.ds(base, CHUNK)], idx_bufs[b], semI
        ).wait()
        if j > 0:
            pltpu.make_async_copy(
                w8_hbm.at[pl.ds(0, CHUNK)], row_bufs[1 - b], row_sems[1 - b]
            ).wait()
        pltpu.async_copy(w8_hbm.at[idx_bufs[b]], row_bufs[b], row_sems[b])
        if j + 1 < NCH:
            pltpu.async_copy(
                idx_hbm.at[pl.ds(base + (j + 1) * CHUNK, CHUNK)],
                idx_bufs[1 - b], semI,
            )
        if j > 0:
            extract_and_store(row_bufs[1 - b], j - 1)
    bl = (NCH - 1) & 1
    pltpu.make_async_copy(
        w8_hbm.at[pl.ds(0, CHUNK)], row_bufs[bl], row_sems[bl]
    ).wait()
    extract_and_store(row_bufs[bl], NCH - 1)


def _asm_body(in_ref, out_ref):
    a = in_ref[...].reshape(128, 8, 128)
    a = jnp.transpose(a, (1, 0, 2))
    out_ref[0] = a.reshape(8, BATCH)


_assemble = pl.pallas_call(
    _asm_body,
    grid=(EMBED_DIM, HIST // 8),
    in_specs=[pl.BlockSpec((8 * BATCH,), lambda d, r: (d * (HIST // 8) + r,))],
    out_specs=pl.BlockSpec((1, 8, BATCH), lambda d, r: (d, r, 0)),
    out_shape=jax.ShapeDtypeStruct((EMBED_DIM, HIST, BATCH), jnp.float32),
)


def kernel(x, W):
    x1f = (jnp.transpose(x).reshape(HIST // 8, 8, BATCH // 128, 128)
           .transpose(0, 2, 1, 3).reshape(N_TOTAL))
    wt = jnp.transpose(W).reshape(EMBED_DIM * N_VOCAB)
    w8 = _prep(wt).reshape(N_VOCAB, PAD_DIM)
    planes = _gather(x1f, w8)
    asm = _assemble(planes)
    return jnp.transpose(asm, (2, 1, 0))
